# baseline (speedup 1.0000x reference)
import functools

import jax
import jax.numpy as jnp
from jax import lax
from jax.experimental import pallas as pl
from jax.experimental.pallas import tpu as pltpu
from jax.experimental.pallas import tpu_sc as plsc

_CHUNK = 128


@functools.cache
def _build_gather(B, V, D, num_cores, num_subcores):
    nw = num_cores * num_subcores
    assert B % (nw * _CHUNK) == 0
    rows_per_w = B // nw
    chunks_per_w = rows_per_w // _CHUNK

    mesh = plsc.VectorSubcoreMesh(core_axis_name="c", subcore_axis_name="s")

    @functools.partial(
        pl.kernel,
        mesh=mesh,
        out_type=jax.ShapeDtypeStruct((B, D), jnp.float32),
        scratch_types=[
            pltpu.VMEM((chunks_per_w, _CHUNK), jnp.int32),
            pltpu.VMEM((_CHUNK, D), jnp.float32),
            pltpu.SemaphoreType.DMA,
        ],
        compiler_params=pltpu.CompilerParams(use_tc_tiling_on_sc=False),
    )
    def gather(idx_hbm, table_hbm, out_hbm, idx_v, rows_v, sem):
        wid = lax.axis_index("s") * num_cores + lax.axis_index("c")
        base_chunk = wid * chunks_per_w
        pltpu.sync_copy(idx_hbm.at[pl.ds(base_chunk, chunks_per_w)], idx_v)

        def body(j, carry):
            pltpu.async_copy(table_hbm.at[idx_v.at[j]], rows_v, sem).wait()
            pltpu.sync_copy(
                rows_v, out_hbm.at[pl.ds((base_chunk + j) * _CHUNK, _CHUNK)]
            )
            return carry

        lax.fori_loop(0, chunks_per_w, body, 0)

    return gather


def kernel(x, table):
    batch, seq = x.shape
    V, D = table.shape
    B = batch * seq
    info = plsc.get_sparse_core_info()
    xf = x.reshape(B // _CHUNK, _CHUNK).astype(jnp.int32)
    out = _build_gather(B, V, D, info.num_cores, info.num_subcores)(xf, table)
    return out.reshape(batch, seq, D)


# speedup vs baseline: 1.1155x; 1.1155x over previous
import functools

import jax
import jax.numpy as jnp
from jax import lax
from jax.experimental import pallas as pl
from jax.experimental.pallas import tpu as pltpu
from jax.experimental.pallas import tpu_sc as plsc

_CHUNK = 128
_NBUF = 4


@functools.cache
def _build_gather(B, V, D, num_cores, num_subcores):
    nw = num_cores * num_subcores
    assert B % (nw * _CHUNK) == 0
    rows_per_w = B // nw
    chunks_per_w = rows_per_w // _CHUNK
    assert chunks_per_w % _NBUF == 0

    mesh = plsc.VectorSubcoreMesh(core_axis_name="c", subcore_axis_name="s")

    scratch = (
        [pltpu.VMEM((chunks_per_w, _CHUNK), jnp.int32)]
        + [pltpu.VMEM((_CHUNK, D), jnp.float32) for _ in range(_NBUF)]
        + [pltpu.SemaphoreType.DMA for _ in range(2 * _NBUF)]
    )

    @functools.partial(
        pl.kernel,
        mesh=mesh,
        out_type=jax.ShapeDtypeStruct((B, D), jnp.float32),
        scratch_types=scratch,
        compiler_params=pltpu.CompilerParams(use_tc_tiling_on_sc=False),
    )
    def gather(idx_hbm, table_hbm, out_hbm, idx_v, *bufs_and_sems):
        bufs = bufs_and_sems[:_NBUF]
        sems_g = bufs_and_sems[_NBUF : 2 * _NBUF]
        sems_s = bufs_and_sems[2 * _NBUF :]

        wid = lax.axis_index("s") * num_cores + lax.axis_index("c")
        base_chunk = wid * chunks_per_w
        pltpu.sync_copy(idx_hbm.at[pl.ds(base_chunk, chunks_per_w)], idx_v)

        def gather_copy(j, b):
            return pltpu.make_async_copy(
                table_hbm.at[idx_v.at[j]], bufs[b], sems_g[b]
            )

        def scatter_copy(j, b):
            return pltpu.make_async_copy(
                bufs[b],
                out_hbm.at[pl.ds((base_chunk + j) * _CHUNK, _CHUNK)],
                sems_s[b],
            )

        for b in range(_NBUF):
            gather_copy(b, b).start()

        @pl.loop(0, chunks_per_w - _NBUF, step=_NBUF)
        def _body(jo):
            for b in range(_NBUF):
                j = jo + b
                gather_copy(j, b).wait()
                scatter_copy(j, b).start()
                scatter_copy(j, b).wait()
                gather_copy(j + _NBUF, b).start()

        for b in range(_NBUF):
            j = chunks_per_w - _NBUF + b
            gather_copy(j, b).wait()
            scatter_copy(j, b).start()
            scatter_copy(j, b).wait()

    return gather


def kernel(x, table):
    batch, seq = x.shape
    V, D = table.shape
    B = batch * seq
    info = plsc.get_sparse_core_info()
    xf = x.reshape(B // _CHUNK, _CHUNK).astype(jnp.int32)
    out = _build_gather(B, V, D, info.num_cores, info.num_subcores)(xf, table)
    return out.reshape(batch, seq, D)
